# Initial kernel scaffold; baseline (speedup 1.0000x reference)
import jax
import jax.numpy as jnp
from jax.experimental import pallas as pl


def kernel(edge_index, edge_type, edge_norm, subj, rel, init_embed, init_rel, w_in1, w_out1, w_loop1, w_rel1, loop_rel1, bias1, gamma1, beta1, w_in2, w_out2, w_loop2, w_rel2, loop_rel2, bias2, gamma2, beta2):
    raise NotImplementedError("write your pallas kernel here")


# speedup vs baseline: 3.0774x; 3.0774x over previous
import functools

import jax
import jax.numpy as jnp
from jax import lax
from jax.experimental import pallas as pl
from jax.experimental.pallas import tpu as pltpu
from jax.experimental.pallas import tpu_sc as plsc

NC = 2
NS = 16
L = 16
D = 128

EDGE_BLK = 80


def _edge_body(x_hbm, r_hbm, src_hbm, dst_hbm, ety_hbm, enorm_hbm, out_hbm,
               srcidx_v, etyidx_v, dstidx_v, enorm_v, xrows_v, rrows_v,
               zero_v, acc_sh, sem_x, sem_r):
    cid = lax.axis_index("c")
    tid = lax.axis_index("s")
    n_ent = x_hbm.shape[0]
    half = src_hbm.shape[0] // 2
    per_tile = half // NS
    n_blk = per_tile // EDGE_BLK
    rows_per_tile = (n_ent // NS) // 8 * 8
    tail_rows = n_ent - NS * rows_per_tile
    zrows = zero_v.shape[0]

    zeros16 = jnp.zeros((L,), jnp.float32)

    def _zrow(i, carry):
        for k in range(D // L):
            zero_v[i, pl.ds(k * L, L)] = zeros16
        return carry

    lax.fori_loop(0, zrows, _zrow, 0)
    for z in range(rows_per_tile // zrows):
        pltpu.sync_copy(zero_v, acc_sh.at[pl.ds(tid * rows_per_tile + z * zrows, zrows)])
    if tail_rows:
        @pl.when(tid == NS - 1)
        def _zero_tail():
            pltpu.sync_copy(zero_v.at[pl.ds(0, tail_rows)],
                            acc_sh.at[pl.ds(NS * rows_per_tile, tail_rows)])
    plsc.subcore_barrier()

    def _block(j, carry):
        base = cid * half + tid * per_tile + j * EDGE_BLK
        pltpu.sync_copy(src_hbm.at[pl.ds(base, EDGE_BLK)], srcidx_v)
        pltpu.sync_copy(ety_hbm.at[pl.ds(base, EDGE_BLK)], etyidx_v)
        pltpu.sync_copy(dst_hbm.at[pl.ds(base, EDGE_BLK)], dstidx_v)
        pltpu.sync_copy(enorm_hbm.at[pl.ds(base, EDGE_BLK)], enorm_v)
        cp_x = pltpu.async_copy(x_hbm.at[srcidx_v], xrows_v, sem_x)
        cp_r = pltpu.async_copy(r_hbm.at[etyidx_v], rrows_v, sem_r)
        cp_x.wait()
        cp_r.wait()

        def _chunk(i16, c):
            en16 = enorm_v[pl.ds(i16 * L, L)]
            for j in range(L):
                i = i16 * L + j
                en = jnp.broadcast_to(en16[j:j + 1], (L,))
                for k in range(D // L):
                    xv = xrows_v[i, pl.ds(k * L, L)]
                    rv = rrows_v[i, pl.ds(k * L, L)]
                    xrows_v[i, pl.ds(k * L, L)] = xv * rv * en
            return c

        lax.fori_loop(0, EDGE_BLK // L, _chunk, 0)
        pltpu.sync_copy(xrows_v, acc_sh.at[dstidx_v], add=True)
        return carry

    lax.fori_loop(0, n_blk, _block, 0)
    plsc.subcore_barrier()

    pltpu.sync_copy(acc_sh.at[pl.ds(tid * rows_per_tile, rows_per_tile)],
                    out_hbm.at[cid, pl.ds(tid * rows_per_tile, rows_per_tile)])
    if tail_rows:
        @pl.when(tid == NS - 1)
        def _write_tail():
            pltpu.sync_copy(acc_sh.at[pl.ds(NS * rows_per_tile, tail_rows)],
                            out_hbm.at[cid, pl.ds(NS * rows_per_tile, tail_rows)])


def _sc_edge_phase(x, r, src, dst, ety, enorm):
    n_ent = x.shape[0]
    zrows = 208
    run = pl.kernel(
        _edge_body,
        out_type=jax.ShapeDtypeStruct((NC, n_ent, D), jnp.float32),
        mesh=plsc.VectorSubcoreMesh(core_axis_name="c", subcore_axis_name="s"),
        scratch_types=[
            pltpu.VMEM((EDGE_BLK,), jnp.int32),
            pltpu.VMEM((EDGE_BLK,), jnp.int32),
            pltpu.VMEM((EDGE_BLK,), jnp.int32),
            pltpu.VMEM((EDGE_BLK,), jnp.float32),
            pltpu.VMEM((EDGE_BLK, D), jnp.float32),
            pltpu.VMEM((EDGE_BLK, D), jnp.float32),
            pltpu.VMEM((zrows, D), jnp.float32),
            pltpu.VMEM_SHARED((n_ent, D), jnp.float32),
            pltpu.SemaphoreType.DMA,
            pltpu.SemaphoreType.DMA,
        ],
    )
    out = run(x, r, src, dst, ety, enorm)
    return out[0], out[1]


def _dense_body(ain_ref, aout_ref, x_ref, r_ref, w_in_ref, w_out_ref,
                w_loop_ref, w_rel_ref, loop_rel_ref, bias_ref, gamma_ref,
                beta_ref, xo_ref, ro_ref):
    agg = jnp.dot(ain_ref[...], w_in_ref[...], preferred_element_type=jnp.float32)
    agg = agg + jnp.dot(aout_ref[...], w_out_ref[...], preferred_element_type=jnp.float32)
    loop = jnp.dot(x_ref[...] * loop_rel_ref[...], w_loop_ref[...],
                   preferred_element_type=jnp.float32)
    h = agg + loop * (1.0 / 3.0) + bias_ref[...]
    mean = jnp.mean(h, axis=0, keepdims=True)
    var = jnp.mean((h - mean) ** 2, axis=0, keepdims=True)
    hn = gamma_ref[...] * (h - mean) * lax.rsqrt(var + 1e-5) + beta_ref[...]
    xo_ref[...] = jnp.tanh(hn)
    ro_ref[...] = jnp.dot(r_ref[...], w_rel_ref[...], preferred_element_type=jnp.float32)


def _tc_dense_phase(a_in, a_out, x, r, w_in, w_out, w_loop, w_rel, loop_rel,
                    bias, gamma, beta):
    n_ent = x.shape[0]
    n_rel = r.shape[0]
    xo, ro = pl.pallas_call(
        _dense_body,
        out_shape=(jax.ShapeDtypeStruct((n_ent, D), jnp.float32),
                   jax.ShapeDtypeStruct((n_rel, D), jnp.float32)),
    )(a_in, a_out, x, r, w_in, w_out, w_loop, w_rel, loop_rel,
      bias.reshape(1, D), gamma.reshape(1, D), beta.reshape(1, D))
    return xo, ro


def _gather_body(x_hbm, r_hbm, subj_hbm, rel_hbm, sub_out, rel_out,
                 idx_v, rows_v, sem):
    cid = lax.axis_index("c")
    tid = lax.axis_index("s")
    wid = tid * NC + cid
    b = sub_out.shape[0] // (NC * NS)
    base = wid * b
    pltpu.sync_copy(subj_hbm.at[pl.ds(base, b)], idx_v)
    pltpu.async_copy(x_hbm.at[idx_v], rows_v, sem).wait()
    pltpu.sync_copy(rows_v, sub_out.at[pl.ds(base, b)])
    pltpu.sync_copy(rel_hbm.at[pl.ds(base, b)], idx_v)
    pltpu.async_copy(r_hbm.at[idx_v], rows_v, sem).wait()
    pltpu.sync_copy(rows_v, rel_out.at[pl.ds(base, b)])


def _sc_gather_phase(x, r, subj, rel):
    batch = subj.shape[0]
    b = batch // (NC * NS)
    run = pl.kernel(
        _gather_body,
        out_type=(jax.ShapeDtypeStruct((batch, D), jnp.float32),
                  jax.ShapeDtypeStruct((batch, D), jnp.float32)),
        mesh=plsc.VectorSubcoreMesh(core_axis_name="c", subcore_axis_name="s"),
        scratch_types=[
            pltpu.VMEM((b,), jnp.int32),
            pltpu.VMEM((b, D), jnp.float32),
            pltpu.SemaphoreType.DMA,
        ],
    )
    return run(x, r, subj, rel)


def kernel(edge_index, edge_type, edge_norm, subj, rel, init_embed, init_rel,
           w_in1, w_out1, w_loop1, w_rel1, loop_rel1, bias1, gamma1, beta1,
           w_in2, w_out2, w_loop2, w_rel2, loop_rel2, bias2, gamma2, beta2):
    src, dst = edge_index[0], edge_index[1]

    a_in1, a_out1 = _sc_edge_phase(init_embed, init_rel, src, dst, edge_type, edge_norm)
    x1, r1 = _tc_dense_phase(a_in1, a_out1, init_embed, init_rel,
                             w_in1, w_out1, w_loop1, w_rel1, loop_rel1,
                             bias1, gamma1, beta1)

    a_in2, a_out2 = _sc_edge_phase(x1, r1, src, dst, edge_type, edge_norm)
    x2, r2 = _tc_dense_phase(a_in2, a_out2, x1, r1,
                             w_in2, w_out2, w_loop2, w_rel2, loop_rel2,
                             bias2, gamma2, beta2)

    sub_emb, rel_emb = _sc_gather_phase(x2, r2, subj, rel)
    return (sub_emb, rel_emb, x2)
